# Initial kernel scaffold; baseline (speedup 1.0000x reference)
import jax
import jax.numpy as jnp
from jax.experimental import pallas as pl


def kernel(seqs, embedding, Wl, Wr, b):
    raise NotImplementedError("write your pallas kernel here")


# speedup vs baseline: 5.5143x; 5.5143x over previous
import functools

import jax
import jax.numpy as jnp
from jax import lax
from jax.experimental import pallas as pl
from jax.experimental.pallas import tpu as pltpu
from jax.experimental.pallas import tpu_sc as plsc

C = 8
EE = 32
E = 256
L = 512
NCHUNK = E // 16
EPS = 1e-8


def _row1(ref):
    return lambda k: ref[pl.ds(k * 16, 16)]


def _row2(ref, i):
    return lambda k: ref[i, pl.ds(k * 16, 16)]


def _vdot(ga, gb):
    acc = jnp.zeros((16,), jnp.float32)
    for k in range(NCHUNK):
        acc = acc + ga(k) * gb(k)
    return jnp.sum(acc)


def _sqrt_scalar(s):
    sv = jnp.full((16,), jnp.maximum(s, 1e-30), dtype=jnp.float32)
    i = plsc.bitcast(sv, jnp.int32)
    i = jnp.int32(0x5F3759DF) - lax.shift_right_arithmetic(i, 1)
    y = plsc.bitcast(i, jnp.float32)
    half = jnp.float32(0.5)
    threehalf = jnp.float32(1.5)
    y = y * (threehalf - half * sv * y * y)
    y = y * (threehalf - half * sv * y * y)
    n = sv * y
    n = half * (n + sv / n)
    return jnp.max(n)


def _vdiv(num, den):
    return jnp.full((16,), num, dtype=jnp.float32) / jnp.full((16,), den, dtype=jnp.float32)


def _store_cos(cos_v, idx, val, lane):
    iv = jnp.full((16,), idx, dtype=jnp.int32)
    vv = val if getattr(val, "shape", ()) == (16,) else jnp.full((16,), val, dtype=jnp.float32)
    plsc.store_scatter(cos_v, [iv], vv, mask=lane == 0)


def _rne_bf16(v):
    xi = plsc.bitcast(v, jnp.int32)
    rb = xi + jnp.int32(0x7FFF) + lax.bitwise_and(lax.shift_right_logical(xi, 16), jnp.int32(1))
    rb = lax.bitwise_and(rb, jnp.int32(-65536))
    return plsc.bitcast(rb, jnp.float32)


def _tanh16(x):
    return jnp.float32(1.0) - jnp.float32(2.0) / (jnp.exp(jnp.float32(2.0) * x) + jnp.float32(1.0))


def _banyan_body(seqs_hbm, emb_hbm, wl_hbm, wr_hbm, b_hbm, out_hbm,
                 ws_sh, idx_v, rows_v, wl_v, wr_v, b_v,
                 cos_v, nrm_v, nxt_v, prv_v,
                 rowl_v, rowr_v, rowp_v, rown_v, sem):
    is_lead = (lax.axis_index("c") == 0) & (lax.axis_index("s") == 0)

    @pl.when(is_lead)
    def _():
        neg_inf = jnp.float32(-jnp.inf)
        eps = jnp.float32(EPS)
        lane = jax.lax.iota(jnp.int32, 16)

        pltpu.sync_copy(wl_hbm, wl_v)
        pltpu.sync_copy(wr_hbm, wr_v)
        pltpu.sync_copy(b_hbm, b_v)
        for j in range(4):
            pltpu.sync_copy(seqs_hbm.at[pl.ds(j * 128, 128)], idx_v.at[j])

        for j in range(4):
            base = j * 128
            pltpu.async_copy(emb_hbm.at[idx_v.at[j]], rows_v, sem).wait()
            pltpu.sync_copy(rows_v, ws_sh.at[pl.ds(base, 128)])

            def nbody(i, _):
                s = _vdot(_row2(rows_v, i), _row2(rows_v, i))
                nrm_v[base + i] = jnp.maximum(_sqrt_scalar(s), eps)
                return 0

            lax.fori_loop(0, 128, nbody, 0)

            def cbody(i, _):
                d = _vdot(_row2(rows_v, i), _row2(rows_v, i + 1))
                _store_cos(cos_v, base + i, _vdiv(d, nrm_v[base + i] * nrm_v[base + i + 1]), lane)
                return 0

            lax.fori_loop(0, 127, cbody, 0)

            if j > 0:
                d = _vdot(_row1(rowl_v), _row2(rows_v, 0))
                _store_cos(cos_v, jnp.int32(base - 1), _vdiv(d, nrm_v[base - 1] * nrm_v[base]), lane)
            if j < 3:
                for k in range(NCHUNK):
                    rowl_v[pl.ds(k * 16, 16)] = rows_v[127, pl.ds(k * 16, 16)]

        _store_cos(cos_v, jnp.int32(L - 1), neg_inf, lane)

        def ibody(i, _):
            nxt_v[i] = i + 1
            prv_v[i] = i - 1
            return 0

        lax.fori_loop(0, L, ibody, 0)

        def step(t, _root):
            best_v = cos_v[pl.ds(0, 16)]
            best_i = lane
            for k in range(1, 32):
                ch = cos_v[pl.ds(k * 16, 16)]
                upd = ch > best_v
                best_v = jnp.where(upd, ch, best_v)
                best_i = jnp.where(upd, lane + k * 16, best_i)
            bv = jnp.max(best_v)
            m = jnp.min(jnp.where(best_v == bv, best_i, jnp.int32(L)))

            r = nxt_v[m]
            p = prv_v[m]

            pltpu.sync_copy(ws_sh.at[m], rowl_v)
            pltpu.sync_copy(ws_sh.at[r], rowr_v)

            sq = jnp.zeros((16,), jnp.float32)
            for c in range(C):
                a0 = b_v[pl.ds(c * EE, 16)]
                a1 = b_v[pl.ds(c * EE + 16, 16)]
                for ke in range(2):
                    lvec = _rne_bf16(rowl_v[pl.ds(c * EE + ke * 16, 16)])
                    rvec = _rne_bf16(rowr_v[pl.ds(c * EE + ke * 16, 16)])
                    for ee in range(16):
                        row = c * EE + ke * 16 + ee
                        lv = lvec[ee]
                        rv = rvec[ee]
                        a0 = a0 + lv * wl_v[row, pl.ds(0, 16)] + rv * wr_v[row, pl.ds(0, 16)]
                        a1 = a1 + lv * wl_v[row, pl.ds(16, 16)] + rv * wr_v[row, pl.ds(16, 16)]
                par0 = _tanh16(a0)
                par1 = _tanh16(a1)
                rowp_v[pl.ds(c * EE, 16)] = par0
                rowp_v[pl.ds(c * EE + 16, 16)] = par1
                sq = sq + par0 * par0 + par1 * par1
            nr = jnp.maximum(_sqrt_scalar(jnp.sum(sq)), eps)

            pltpu.sync_copy(rowp_v, ws_sh.at[r])
            nrm_v[r] = nr
            prv_v[r] = p
            _store_cos(cos_v, m, neg_inf, lane)

            @pl.when(p >= 0)
            def _():
                nxt_v[p] = r
                pltpu.sync_copy(ws_sh.at[p], rown_v)
                d = _vdot(_row1(rown_v), _row1(rowp_v))
                _store_cos(cos_v, p, _vdiv(d, nrm_v[p] * nr), lane)

            n2 = nxt_v[r]

            @pl.when(n2 < L)
            def _():
                pltpu.sync_copy(ws_sh.at[n2], rown_v)
                d = _vdot(_row1(rowp_v), _row1(rown_v))
                _store_cos(cos_v, r, _vdiv(d, nr * nrm_v[n2]), lane)

            @pl.when(n2 >= L)
            def _():
                _store_cos(cos_v, r, neg_inf, lane)

            return r

        root = lax.fori_loop(0, L - 1, step, jnp.int32(0))
        pltpu.sync_copy(ws_sh.at[root], out_hbm)


@jax.jit
def _banyan_sc(seqs, embedding, wl2, wr2, b2):
    mesh = plsc.VectorSubcoreMesh(core_axis_name="c", subcore_axis_name="s")
    return pl.kernel(
        _banyan_body,
        out_type=jax.ShapeDtypeStruct((E,), jnp.float32),
        mesh=mesh,
        compiler_params=pltpu.CompilerParams(needs_layout_passes=False),
        scratch_types=[
            pltpu.VMEM_SHARED((L, E), jnp.float32),
            pltpu.VMEM((4, 128), jnp.int32),
            pltpu.VMEM((128, E), jnp.float32),
            pltpu.VMEM((C * EE, EE), jnp.float32),
            pltpu.VMEM((C * EE, EE), jnp.float32),
            pltpu.VMEM((E,), jnp.float32),
            pltpu.VMEM((L,), jnp.float32),
            pltpu.SMEM((L,), jnp.float32),
            pltpu.SMEM((L,), jnp.int32),
            pltpu.SMEM((L,), jnp.int32),
            pltpu.VMEM((E,), jnp.float32),
            pltpu.VMEM((E,), jnp.float32),
            pltpu.VMEM((E,), jnp.float32),
            pltpu.VMEM((E,), jnp.float32),
            pltpu.SemaphoreType.DMA,
        ],
    )(seqs, embedding, wl2, wr2, b2)


def kernel(seqs, embedding, Wl, Wr, b):
    seqs = seqs.astype(jnp.int32)
    wl2 = Wl.reshape(C * EE, EE).astype(jnp.bfloat16).astype(jnp.float32)
    wr2 = Wr.reshape(C * EE, EE).astype(jnp.bfloat16).astype(jnp.float32)
    b2 = b.reshape(E)
    return _banyan_sc(seqs, embedding, wl2, wr2, b2)
